# baseline (speedup 1.0000x reference)
import jax
import jax.numpy as jnp
from jax.experimental import pallas as pl
from jax.experimental.pallas import tpu as pltpu

B_, A_, NN_, F_, RES_, T_ = 16, 128, 32, 128, 20, 3
CUTOFF = 5.0
K_COUL = 332.063711
AN = A_ * NN_
f32 = jnp.float32


def _sig(x):
    return 1.0 / (1.0 + jnp.exp(-x))


def _silu(x):
    return x * _sig(x)


def _dsilu(x):
    s = _sig(x)
    return s * (1.0 + x * (1.0 - s))


def _mm(a, b):
    return jax.lax.dot(a, b, preferred_element_type=f32)


def _mmx(a, b):
    bf16 = jnp.bfloat16
    hi = b.astype(bf16).astype(f32)
    r1 = b - hi
    mid = r1.astype(bf16).astype(f32)
    lo = (r1 - mid).astype(bf16).astype(f32)
    return _mm(a, hi) + _mm(a, mid) + _mm(a, lo)


def _mayer_body(R_ref, Zc_ref, N3_ref, Nrow_ref,
                embed_r, Wf1_r, bf1_r, Wf2_r, bf2_r, Wu_r, bu_r,
                Wa_r, Wp_r, Wf1T_r, Wf2T_r, WuT_r, WaT_r, WpT_r,
                E_ref, F_ref, Q_ref, Bm_ref, D_ref,
                hs_ref):
    R = R_ref[0]
    Zc = Zc_ref[0]
    N3 = N3_ref[0]
    Nrow = Nrow_ref[0]

    iota3 = jax.lax.broadcasted_iota(jnp.int32, (A_, NN_, A_), 2)
    G = (N3[:, :, None] == iota3).astype(f32).reshape(AN, A_)
    row_a = jax.lax.broadcasted_iota(jnp.int32, (A_, AN), 0)
    GT = (Nrow == row_a).astype(f32)

    def _rep(x):
        w = x.shape[1]
        return jnp.broadcast_to(x[:, None, :], (A_, NN_, w)).reshape(AN, w)

    def _seg(x):
        w = x.shape[1]
        return jnp.sum(x.reshape(A_, NN_, w), axis=1)

    D2 = jnp.sum((_rep(R) - _mmx(G, R)) ** 2, axis=1, keepdims=True)
    D = jnp.sqrt(D2 + 1e-12)
    centers = (jax.lax.broadcasted_iota(jnp.int32, (1, RES_), 1).astype(f32)
               * (CUTOFF / (RES_ - 1)))
    rbf = jnp.exp(-10.0 * (D - centers) ** 2)
    fc = 0.5 * (jnp.cos(jnp.pi * jnp.clip(D / CUTOFF, 0.0, 1.0)) + 1.0) \
        * (D < CUTOFF).astype(f32)

    lane_z = jax.lax.broadcasted_iota(jnp.int32, (A_, 100), 1)
    onehotZ = (Zc == lane_z).astype(f32)

    def fwd_net(inet, _):
        h = _mm(onehotZ, embed_r[inet])
        hs_ref[inet, 0] = h

        def layer(t, h):
            hj = _mm(G, h)
            pre = _mm(rbf, Wf1_r[inet, t]) + bf1_r[inet, t]
            W = _mm(_silu(pre), Wf2_r[inet, t]) + bf2_r[inet, t]
            m = _seg(hj * W * fc)
            u = _mm(m, Wu_r[inet, t]) + bu_r[inet, t]
            h = h + _silu(u)
            hs_ref[inet, t + 1] = h
            return h

        jax.lax.fori_loop(0, T_, layer, h)
        return 0

    jax.lax.fori_loop(0, 2, fwd_net, 0)

    h3c = hs_ref[0, T_]
    h3d = hs_ref[1, T_]
    Q = _mm(h3c, Wa_r[0])
    Bm = _mm(_rep(h3c) * _mm(G, h3c), Wp_r[0])

    qi = _rep(Q)
    qj = _mmx(G, Q)
    mask = (D > 1e-6).astype(f32)
    D_inv = mask * (1.0 / D)
    E_coul = 0.5 * K_COUL * jnp.sum(D_inv * qi * qj, axis=(0, 1), keepdims=True)
    E_bond = -0.25 * K_COUL * jnp.sum(D_inv * Bm * Bm, axis=(0, 1), keepdims=True)
    dE = jnp.sum(_mm(h3d, Wa_r[1]), axis=(0, 1), keepdims=True)
    E = E_coul + E_bond + dE

    gQ = 0.5 * K_COUL * (_seg(D_inv * qj) + _mmx(GT, D_inv * qi))
    gBm = -0.5 * K_COUL * D_inv * Bm
    gDinv = 0.5 * K_COUL * qi * qj - 0.25 * K_COUL * Bm * Bm
    gD = -gDinv * D_inv * D_inv * mask

    ones_A1 = jnp.ones((A_, 1), f32)
    zeros_AN1 = jnp.zeros((AN, 1), f32)

    def bwd_net(inet, carry):
        grbf_t, gfc_t = carry
        is_chg = (inet == 0)
        gAi = jnp.where(is_chg, gQ, ones_A1)
        gPij = jnp.where(is_chg, gBm, zeros_AN1)
        h3 = hs_ref[inet, T_]
        hj3 = _mm(G, h3)
        hrep = _rep(h3)
        gh0 = gAi * WaT_r[inet]
        WpT = WpT_r[inet]
        ghrep = gPij * (hj3 * WpT)
        ghj3 = gPij * (hrep * WpT)
        gh0 = gh0 + _seg(ghrep) + _mm(GT, ghj3)

        def layer(i, carry):
            gh, grbf, gfc = carry
            t = T_ - 1 - i
            h_in = hs_ref[inet, t]
            hj = _mm(G, h_in)
            pre = _mm(rbf, Wf1_r[inet, t]) + bf1_r[inet, t]
            s1 = _silu(pre)
            W = _mm(s1, Wf2_r[inet, t]) + bf2_r[inet, t]
            m = _seg(hj * W * fc)
            u = _mm(m, Wu_r[inet, t]) + bu_r[inet, t]
            gu = gh * _dsilu(u)
            gm = _mm(gu, WuT_r[inet, t])
            gmr = _rep(gm)
            ghj = gmr * W * fc
            gW = gmr * hj * fc
            gfc = gfc + jnp.sum(gmr * hj * W, axis=1, keepdims=True)
            gpre = _mm(gW, Wf2T_r[inet, t]) * _dsilu(pre)
            grbf = grbf + _mm(gpre, Wf1T_r[inet, t])
            gh = gh + _mm(GT, ghj)
            return gh, grbf, gfc

        _, grbf_t, gfc_t = jax.lax.fori_loop(
            0, T_, layer, (gh0, grbf_t, gfc_t))
        return grbf_t, gfc_t

    grbf, gfc = jax.lax.fori_loop(
        0, 2, bwd_net, (jnp.zeros((AN, RES_), f32), zeros_AN1))

    gD = gD + jnp.sum(grbf * rbf * (-20.0 * (D - centers)), axis=1, keepdims=True)
    gD = gD + gfc * (-0.5 * jnp.pi / CUTOFF) * jnp.sin(
        jnp.pi * jnp.clip(D / CUTOFF, 0.0, 1.0)) * (D < CUTOFF).astype(f32)

    diff = _rep(R) - _mmx(G, R)
    gdiff = (gD / D) * diff
    gR = _seg(gdiff) - _mmx(GT, gdiff)

    row_n = jax.lax.broadcasted_iota(jnp.int32, (AN, NN_), 0)
    Emat = (row_n % NN_ ==
            jax.lax.broadcasted_iota(jnp.int32, (AN, NN_), 1)).astype(f32)
    E_ref[0] = E
    F_ref[0] = -gR
    Q_ref[0] = Q
    Bm_ref[0] = _seg(Bm * Emat)
    D_ref[0] = _seg(D * Emat)


def _run(interpret, R, Z, N,
         chg_embed, chg_Wf1, chg_bf1, chg_Wf2, chg_bf2, chg_Wu, chg_bu,
         chg_Wa, chg_Wp,
         dlt_embed, dlt_Wf1, dlt_bf1, dlt_Wf2, dlt_bf2, dlt_Wu, dlt_bu,
         dlt_Wa, dlt_Wp):
    Zc = Z.reshape(B_, A_, 1).astype(jnp.int32)
    N3 = N.reshape(B_, A_, NN_).astype(jnp.int32)
    Nrow = N.reshape(B_, 1, AN).astype(jnp.int32)

    def st(c, d):
        return jnp.stack([c, d]).astype(f32)

    embed2 = st(chg_embed, dlt_embed)
    Wf12 = st(chg_Wf1, dlt_Wf1)
    bf12 = st(chg_bf1, dlt_bf1).reshape(2, T_, 1, F_)
    Wf22 = st(chg_Wf2, dlt_Wf2)
    bf22 = st(chg_bf2, dlt_bf2).reshape(2, T_, 1, F_)
    Wu2 = st(chg_Wu, dlt_Wu)
    bu2 = st(chg_bu, dlt_bu).reshape(2, T_, 1, F_)
    Wa2 = st(chg_Wa, dlt_Wa)
    Wp2 = st(chg_Wp, dlt_Wp)
    Wf1T2 = jnp.transpose(Wf12, (0, 1, 3, 2))
    Wf2T2 = jnp.transpose(Wf22, (0, 1, 3, 2))
    WuT2 = jnp.transpose(Wu2, (0, 1, 3, 2))
    WaT2 = Wa2.reshape(2, 1, F_)
    WpT2 = Wp2.reshape(2, 1, F_)

    wargs = (embed2, Wf12, bf12, Wf22, bf22, Wu2, bu2, Wa2, Wp2,
             Wf1T2, Wf2T2, WuT2, WaT2, WpT2)

    def bspec(shape):
        return pl.BlockSpec((1,) + shape, lambda b: (b, 0, 0))

    def wspec(arr):
        nd = arr.ndim
        return pl.BlockSpec(arr.shape, lambda b, _n=nd: (0,) * _n)

    in_specs = [bspec((A_, 3)), bspec((A_, 1)), bspec((A_, NN_)), bspec((1, AN))]
    in_specs += [wspec(a) for a in wargs]

    out_shapes = (jax.ShapeDtypeStruct((B_, 1, 1), f32),
                  jax.ShapeDtypeStruct((B_, A_, 3), f32),
                  jax.ShapeDtypeStruct((B_, A_, 1), f32),
                  jax.ShapeDtypeStruct((B_, A_, NN_), f32),
                  jax.ShapeDtypeStruct((B_, A_, NN_), f32))
    out_specs = (bspec((1, 1)), bspec((A_, 3)), bspec((A_, 1)),
                 bspec((A_, NN_)), bspec((A_, NN_)))

    scratch = [pltpu.VMEM((2, T_ + 1, A_, F_), f32)]

    E3, F, Q, Bm, D = pl.pallas_call(
        _mayer_body,
        grid=(B_,),
        in_specs=in_specs,
        out_specs=out_specs,
        out_shape=out_shapes,
        scratch_shapes=scratch,
        compiler_params=pltpu.CompilerParams(
            dimension_semantics=("parallel",)),
        interpret=interpret,
    )(R.astype(f32), Zc, N3, Nrow, *wargs)

    return (E3.reshape(B_, 1), F, Q, Bm, D)


def kernel(R, Z, N,
           chg_embed, chg_Wf1, chg_bf1, chg_Wf2, chg_bf2, chg_Wu, chg_bu,
           chg_Wa, chg_Wp,
           dlt_embed, dlt_Wf1, dlt_bf1, dlt_Wf2, dlt_bf2, dlt_Wu, dlt_bu,
           dlt_Wa, dlt_Wp):
    return _run(False, R, Z, N,
                chg_embed, chg_Wf1, chg_bf1, chg_Wf2, chg_bf2, chg_Wu,
                chg_bu, chg_Wa, chg_Wp,
                dlt_embed, dlt_Wf1, dlt_bf1, dlt_Wf2, dlt_bf2, dlt_Wu,
                dlt_bu, dlt_Wa, dlt_Wp)


# speedup vs baseline: 1.1222x; 1.1222x over previous
import jax
import jax.numpy as jnp
from jax.experimental import pallas as pl
from jax.experimental.pallas import tpu as pltpu

B_, A_, NN_, F_, RES_, T_ = 16, 128, 32, 128, 20, 3
CUTOFF = 5.0
K_COUL = 332.063711
AN = A_ * NN_
f32 = jnp.float32


def _sig(x):
    return 1.0 / (1.0 + jnp.exp(-x))


def _silu(x):
    return x * _sig(x)


def _dsilu(x):
    s = _sig(x)
    return s * (1.0 + x * (1.0 - s))


def _mm(a, b):
    return jax.lax.dot(a, b, preferred_element_type=f32)


def _mmx(a, b):
    bf16 = jnp.bfloat16
    hi = b.astype(bf16).astype(f32)
    r1 = b - hi
    mid = r1.astype(bf16).astype(f32)
    lo = (r1 - mid).astype(bf16).astype(f32)
    return _mm(a, hi) + _mm(a, mid) + _mm(a, lo)


def _mayer_body(R_ref, Zc_ref, N3_ref, Nrow_ref,
                embed_r, Wf1_r, bf1_r, Wf2_r, bf2_r, Wu_r, bu_r,
                Wa_r, Wp_r, Wf1T_r, Wf2T_r, WuT_r, WaT_r, WpT_r,
                E_ref, F_ref, Q_ref, Bm_ref, D_ref,
                hs_ref):
    R = R_ref[0]
    Zc = Zc_ref[0]
    N3 = N3_ref[0]
    Nrow = Nrow_ref[0]

    iota3 = jax.lax.broadcasted_iota(jnp.int32, (A_, NN_, A_), 2)
    G = (N3[:, :, None] == iota3).astype(f32).reshape(AN, A_)
    row_a = jax.lax.broadcasted_iota(jnp.int32, (A_, AN), 0)
    GT = (Nrow == row_a).astype(f32)

    def _rep(x):
        w = x.shape[1]
        return jnp.broadcast_to(x[:, None, :], (A_, NN_, w)).reshape(AN, w)

    def _seg(x):
        w = x.shape[1]
        return jnp.sum(x.reshape(A_, NN_, w), axis=1)

    eye3 = (jax.lax.broadcasted_iota(jnp.int32, (1, NN_, NN_), 1) ==
            jax.lax.broadcasted_iota(jnp.int32, (1, NN_, NN_), 2)).astype(f32)

    def _colify(xm):
        return jnp.sum(xm[:, None, :] * eye3, axis=2,
                       keepdims=True).reshape(AN, 1)

    def _matify(xc):
        return jnp.sum(xc.reshape(A_, NN_, 1) * eye3, axis=1)

    D2 = jnp.sum((_rep(R) - _mmx(G, R)) ** 2, axis=1, keepdims=True)
    D = jnp.sqrt(D2 + 1e-12)
    Dm = _matify(D)
    centers = (jax.lax.broadcasted_iota(jnp.int32, (1, RES_), 1).astype(f32)
               * (CUTOFF / (RES_ - 1)))
    rbf = jnp.exp(-10.0 * (D - centers) ** 2)
    fcm = 0.5 * (jnp.cos(jnp.pi * jnp.clip(Dm / CUTOFF, 0.0, 1.0)) + 1.0) \
        * (Dm < CUTOFF).astype(f32)
    fc = _colify(fcm)

    lane_z = jax.lax.broadcasted_iota(jnp.int32, (A_, 100), 1)
    onehotZ = (Zc == lane_z).astype(f32)

    def fwd_net(inet, _):
        h = _mm(onehotZ, embed_r[inet])
        hs_ref[inet, 0] = h

        def layer(t, h):
            hj = _mm(G, h)
            pre = _mm(rbf, Wf1_r[inet, t]) + bf1_r[inet, t]
            W = _mm(_silu(pre), Wf2_r[inet, t]) + bf2_r[inet, t]
            m = _seg(hj * W * fc)
            u = _mm(m, Wu_r[inet, t]) + bu_r[inet, t]
            h = h + _silu(u)
            hs_ref[inet, t + 1] = h
            return h

        jax.lax.fori_loop(0, T_, layer, h)
        return 0

    jax.lax.fori_loop(0, 2, fwd_net, 0)

    h3c = hs_ref[0, T_]
    h3d = hs_ref[1, T_]
    Q = _mm(h3c, Wa_r[0])
    Bm = _mm(_rep(h3c) * _mm(G, h3c), Wp_r[0])

    qim = jnp.broadcast_to(Q, (A_, NN_))
    qjm = _matify(_mmx(G, Q))
    Bmm = _matify(Bm)
    maskm = (Dm > 1e-6).astype(f32)
    Dinvm = maskm * (1.0 / Dm)
    E_coul = 0.5 * K_COUL * jnp.sum(Dinvm * qim * qjm, axis=(0, 1), keepdims=True)
    E_bond = -0.25 * K_COUL * jnp.sum(Dinvm * Bmm * Bmm, axis=(0, 1), keepdims=True)
    dE = jnp.sum(_mm(h3d, Wa_r[1]), axis=(0, 1), keepdims=True)
    E = E_coul + E_bond + dE

    gQ = 0.5 * K_COUL * (jnp.sum(Dinvm * qjm, axis=1, keepdims=True)
                         + _mmx(GT, _colify(Dinvm * qim)))
    gBm = _colify(-0.5 * K_COUL * Dinvm * Bmm)
    gDinvm = 0.5 * K_COUL * qim * qjm - 0.25 * K_COUL * Bmm * Bmm
    gDm = -gDinvm * Dinvm * Dinvm * maskm

    ones_A1 = jnp.ones((A_, 1), f32)
    zeros_AN1 = jnp.zeros((AN, 1), f32)

    def bwd_net(inet, carry):
        grbf_t, gfc_t = carry
        is_chg = (inet == 0)
        gAi = jnp.where(is_chg, gQ, ones_A1)
        gPij = jnp.where(is_chg, gBm, zeros_AN1)
        h3 = hs_ref[inet, T_]
        hj3 = _mm(G, h3)
        hrep = _rep(h3)
        gh0 = gAi * WaT_r[inet]
        WpT = WpT_r[inet]
        ghrep = gPij * (hj3 * WpT)
        ghj3 = gPij * (hrep * WpT)
        gh0 = gh0 + _seg(ghrep) + _mm(GT, ghj3)

        def layer(i, carry):
            gh, grbf, gfc = carry
            t = T_ - 1 - i
            h_in = hs_ref[inet, t]
            hj = _mm(G, h_in)
            pre = _mm(rbf, Wf1_r[inet, t]) + bf1_r[inet, t]
            s1 = _silu(pre)
            W = _mm(s1, Wf2_r[inet, t]) + bf2_r[inet, t]
            m = _seg(hj * W * fc)
            u = _mm(m, Wu_r[inet, t]) + bu_r[inet, t]
            gu = gh * _dsilu(u)
            gm = _mm(gu, WuT_r[inet, t])
            gmr = _rep(gm)
            ghj = gmr * W * fc
            gW = gmr * hj * fc
            gfc = gfc + jnp.sum(gmr * hj * W, axis=1, keepdims=True)
            gpre = _mm(gW, Wf2T_r[inet, t]) * _dsilu(pre)
            grbf = grbf + _mm(gpre, Wf1T_r[inet, t])
            gh = gh + _mm(GT, ghj)
            return gh, grbf, gfc

        _, grbf_t, gfc_t = jax.lax.fori_loop(
            0, T_, layer, (gh0, grbf_t, gfc_t))
        return grbf_t, gfc_t

    grbf, gfc = jax.lax.fori_loop(
        0, 2, bwd_net, (jnp.zeros((AN, RES_), f32), zeros_AN1))

    gD_rbf = jnp.sum(grbf * rbf * (-20.0 * (D - centers)), axis=1, keepdims=True)
    gDm = gDm + _matify(gD_rbf)
    gDm = gDm + _matify(gfc) * (-0.5 * jnp.pi / CUTOFF) * jnp.sin(
        jnp.pi * jnp.clip(Dm / CUTOFF, 0.0, 1.0)) * (Dm < CUTOFF).astype(f32)

    diff = _rep(R) - _mmx(G, R)
    gdiff = _colify(gDm / Dm) * diff
    gR = _seg(gdiff) - _mmx(GT, gdiff)

    E_ref[0] = E
    F_ref[0] = -gR
    Q_ref[0] = Q
    Bm_ref[0] = Bmm
    D_ref[0] = Dm


def _run(interpret, R, Z, N,
         chg_embed, chg_Wf1, chg_bf1, chg_Wf2, chg_bf2, chg_Wu, chg_bu,
         chg_Wa, chg_Wp,
         dlt_embed, dlt_Wf1, dlt_bf1, dlt_Wf2, dlt_bf2, dlt_Wu, dlt_bu,
         dlt_Wa, dlt_Wp):
    Zc = Z.reshape(B_, A_, 1).astype(jnp.int32)
    N3 = N.reshape(B_, A_, NN_).astype(jnp.int32)
    Nrow = N.reshape(B_, 1, AN).astype(jnp.int32)

    def st(c, d):
        return jnp.stack([c, d]).astype(f32)

    embed2 = st(chg_embed, dlt_embed)
    Wf12 = st(chg_Wf1, dlt_Wf1)
    bf12 = st(chg_bf1, dlt_bf1).reshape(2, T_, 1, F_)
    Wf22 = st(chg_Wf2, dlt_Wf2)
    bf22 = st(chg_bf2, dlt_bf2).reshape(2, T_, 1, F_)
    Wu2 = st(chg_Wu, dlt_Wu)
    bu2 = st(chg_bu, dlt_bu).reshape(2, T_, 1, F_)
    Wa2 = st(chg_Wa, dlt_Wa)
    Wp2 = st(chg_Wp, dlt_Wp)
    Wf1T2 = jnp.transpose(Wf12, (0, 1, 3, 2))
    Wf2T2 = jnp.transpose(Wf22, (0, 1, 3, 2))
    WuT2 = jnp.transpose(Wu2, (0, 1, 3, 2))
    WaT2 = Wa2.reshape(2, 1, F_)
    WpT2 = Wp2.reshape(2, 1, F_)

    wargs = (embed2, Wf12, bf12, Wf22, bf22, Wu2, bu2, Wa2, Wp2,
             Wf1T2, Wf2T2, WuT2, WaT2, WpT2)

    def bspec(shape):
        return pl.BlockSpec((1,) + shape, lambda b: (b, 0, 0))

    def wspec(arr):
        nd = arr.ndim
        return pl.BlockSpec(arr.shape, lambda b, _n=nd: (0,) * _n)

    in_specs = [bspec((A_, 3)), bspec((A_, 1)), bspec((A_, NN_)), bspec((1, AN))]
    in_specs += [wspec(a) for a in wargs]

    out_shapes = (jax.ShapeDtypeStruct((B_, 1, 1), f32),
                  jax.ShapeDtypeStruct((B_, A_, 3), f32),
                  jax.ShapeDtypeStruct((B_, A_, 1), f32),
                  jax.ShapeDtypeStruct((B_, A_, NN_), f32),
                  jax.ShapeDtypeStruct((B_, A_, NN_), f32))
    out_specs = (bspec((1, 1)), bspec((A_, 3)), bspec((A_, 1)),
                 bspec((A_, NN_)), bspec((A_, NN_)))

    scratch = [pltpu.VMEM((2, T_ + 1, A_, F_), f32)]

    E3, F, Q, Bm, D = pl.pallas_call(
        _mayer_body,
        grid=(B_,),
        in_specs=in_specs,
        out_specs=out_specs,
        out_shape=out_shapes,
        scratch_shapes=scratch,
        compiler_params=pltpu.CompilerParams(
            dimension_semantics=("parallel",)),
        interpret=interpret,
    )(R.astype(f32), Zc, N3, Nrow, *wargs)

    return (E3.reshape(B_, 1), F, Q, Bm, D)


def kernel(R, Z, N,
           chg_embed, chg_Wf1, chg_bf1, chg_Wf2, chg_bf2, chg_Wu, chg_bu,
           chg_Wa, chg_Wp,
           dlt_embed, dlt_Wf1, dlt_bf1, dlt_Wf2, dlt_bf2, dlt_Wu, dlt_bu,
           dlt_Wa, dlt_Wp):
    return _run(False, R, Z, N,
                chg_embed, chg_Wf1, chg_bf1, chg_Wf2, chg_bf2, chg_Wu,
                chg_bu, chg_Wa, chg_Wp,
                dlt_embed, dlt_Wf1, dlt_bf1, dlt_Wf2, dlt_bf2, dlt_Wu,
                dlt_bu, dlt_Wa, dlt_Wp)


# speedup vs baseline: 1.1251x; 1.0026x over previous
import jax
import jax.numpy as jnp
from jax.experimental import pallas as pl
from jax.experimental.pallas import tpu as pltpu

B_, A_, NN_, F_, RES_, T_ = 16, 128, 32, 128, 20, 3
CUTOFF = 5.0
K_COUL = 332.063711
AN = A_ * NN_
f32 = jnp.float32


def _sig(x):
    return jax.nn.sigmoid(x)


def _silu(x):
    return x * _sig(x)


def _dsilu(x):
    s = _sig(x)
    return s * (1.0 + x * (1.0 - s))


def _mm(a, b):
    return jax.lax.dot(a, b, preferred_element_type=f32)


def _mmx(a, b):
    bf16 = jnp.bfloat16
    hi = b.astype(bf16).astype(f32)
    r1 = b - hi
    mid = r1.astype(bf16).astype(f32)
    lo = (r1 - mid).astype(bf16).astype(f32)
    return _mm(a, hi) + _mm(a, mid) + _mm(a, lo)


def _mayer_body(R_ref, Zc_ref, N3_ref, Nrow_ref,
                embed_r, Wf1_r, bf1_r, Wf2_r, bf2_r, Wu_r, bu_r,
                Wa_r, Wp_r, Wf1T_r, Wf2T_r, WuT_r, WaT_r, WpT_r,
                E_ref, F_ref, Q_ref, Bm_ref, D_ref,
                hs_ref):
    R = R_ref[0]
    Zc = Zc_ref[0]
    N3 = N3_ref[0]
    Nrow = Nrow_ref[0]

    iota3 = jax.lax.broadcasted_iota(jnp.int32, (A_, NN_, A_), 2)
    G = (N3[:, :, None] == iota3).astype(f32).reshape(AN, A_)
    row_a = jax.lax.broadcasted_iota(jnp.int32, (A_, AN), 0)
    GT = (Nrow == row_a).astype(f32)

    def _rep(x):
        w = x.shape[1]
        return jnp.broadcast_to(x[:, None, :], (A_, NN_, w)).reshape(AN, w)

    def _seg(x):
        w = x.shape[1]
        return jnp.sum(x.reshape(A_, NN_, w), axis=1)

    eye3 = (jax.lax.broadcasted_iota(jnp.int32, (1, NN_, NN_), 1) ==
            jax.lax.broadcasted_iota(jnp.int32, (1, NN_, NN_), 2)).astype(f32)

    def _colify(xm):
        return jnp.sum(xm[:, None, :] * eye3, axis=2,
                       keepdims=True).reshape(AN, 1)

    def _matify(xc):
        return jnp.sum(xc.reshape(A_, NN_, 1) * eye3, axis=1)

    D2 = jnp.sum((_rep(R) - _mmx(G, R)) ** 2, axis=1, keepdims=True)
    D = jnp.sqrt(D2 + 1e-12)
    Dm = _matify(D)
    centers = (jax.lax.broadcasted_iota(jnp.int32, (1, RES_), 1).astype(f32)
               * (CUTOFF / (RES_ - 1)))
    rbf = jnp.exp(-10.0 * (D - centers) ** 2)
    fcm = 0.5 * (jnp.cos(jnp.pi * jnp.clip(Dm / CUTOFF, 0.0, 1.0)) + 1.0) \
        * (Dm < CUTOFF).astype(f32)
    fc = _colify(fcm)

    lane_z = jax.lax.broadcasted_iota(jnp.int32, (A_, 100), 1)
    onehotZ = (Zc == lane_z).astype(f32)

    def fwd_net(inet, _):
        h = _mm(onehotZ, embed_r[inet])
        hs_ref[inet, 0] = h

        def layer(t, h):
            hj = _mm(G, h)
            pre = _mm(rbf, Wf1_r[inet, t]) + bf1_r[inet, t]
            W = _mm(_silu(pre), Wf2_r[inet, t]) + bf2_r[inet, t]
            m = _seg(hj * (W * fc))
            u = _mm(m, Wu_r[inet, t]) + bu_r[inet, t]
            h = h + _silu(u)
            hs_ref[inet, t + 1] = h
            return h

        jax.lax.fori_loop(0, T_, layer, h)
        return 0

    jax.lax.fori_loop(0, 2, fwd_net, 0)

    h3c = hs_ref[0, T_]
    h3d = hs_ref[1, T_]
    Q = _mm(h3c, Wa_r[0])
    Bm = _mm(_rep(h3c) * _mm(G, h3c), Wp_r[0])

    qim = jnp.broadcast_to(Q, (A_, NN_))
    qjm = _matify(_mmx(G, Q))
    Bmm = _matify(Bm)
    maskm = (Dm > 1e-6).astype(f32)
    Dinvm = maskm * (1.0 / Dm)
    E_coul = 0.5 * K_COUL * jnp.sum(Dinvm * qim * qjm, axis=(0, 1), keepdims=True)
    E_bond = -0.25 * K_COUL * jnp.sum(Dinvm * Bmm * Bmm, axis=(0, 1), keepdims=True)
    dE = jnp.sum(_mm(h3d, Wa_r[1]), axis=(0, 1), keepdims=True)
    E = E_coul + E_bond + dE

    gQ = 0.5 * K_COUL * (jnp.sum(Dinvm * qjm, axis=1, keepdims=True)
                         + _mmx(GT, _colify(Dinvm * qim)))
    gBm = _colify(-0.5 * K_COUL * Dinvm * Bmm)
    gDinvm = 0.5 * K_COUL * qim * qjm - 0.25 * K_COUL * Bmm * Bmm
    gDm = -gDinvm * Dinvm * Dinvm * maskm

    ones_A1 = jnp.ones((A_, 1), f32)
    zeros_AN1 = jnp.zeros((AN, 1), f32)

    def bwd_net(inet, carry):
        grbf_t, gfc_t = carry
        is_chg = (inet == 0)
        gAi = jnp.where(is_chg, gQ, ones_A1)
        gPij = jnp.where(is_chg, gBm, zeros_AN1)
        h3 = hs_ref[inet, T_]
        hj3 = _mm(G, h3)
        hrep = _rep(h3)
        gh0 = gAi * WaT_r[inet]
        WpT = WpT_r[inet]
        ghrep = gPij * (hj3 * WpT)
        ghj3 = gPij * (hrep * WpT)
        gh0 = gh0 + _seg(ghrep) + _mm(GT, ghj3)

        def layer(i, carry):
            gh, grbf, gfc = carry
            t = T_ - 1 - i
            h_in = hs_ref[inet, t]
            hj = _mm(G, h_in)
            pre = _mm(rbf, Wf1_r[inet, t]) + bf1_r[inet, t]
            sg = _sig(pre)
            W = _mm(pre * sg, Wf2_r[inet, t]) + bf2_r[inet, t]
            Wfc = W * fc
            m = _seg(hj * Wfc)
            u = _mm(m, Wu_r[inet, t]) + bu_r[inet, t]
            gu = gh * _dsilu(u)
            gm = _mm(gu, WuT_r[inet, t])
            gmr = _rep(gm)
            t1 = gmr * hj
            ghj = gmr * Wfc
            gW = t1 * fc
            gfc = gfc + jnp.sum(t1 * W, axis=1, keepdims=True)
            dsg = sg * (1.0 + pre * (1.0 - sg))
            gpre = _mm(gW, Wf2T_r[inet, t]) * dsg
            grbf = grbf + _mm(gpre, Wf1T_r[inet, t])
            gh = gh + _mm(GT, ghj)
            return gh, grbf, gfc

        _, grbf_t, gfc_t = jax.lax.fori_loop(
            0, T_, layer, (gh0, grbf_t, gfc_t))
        return grbf_t, gfc_t

    grbf, gfc = jax.lax.fori_loop(
        0, 2, bwd_net, (jnp.zeros((AN, RES_), f32), zeros_AN1))

    gD_rbf = jnp.sum(grbf * rbf * (-20.0 * (D - centers)), axis=1, keepdims=True)
    gDm = gDm + _matify(gD_rbf)
    gDm = gDm + _matify(gfc) * (-0.5 * jnp.pi / CUTOFF) * jnp.sin(
        jnp.pi * jnp.clip(Dm / CUTOFF, 0.0, 1.0)) * (Dm < CUTOFF).astype(f32)

    diff = _rep(R) - _mmx(G, R)
    gdiff = _colify(gDm / Dm) * diff
    gR = _seg(gdiff) - _mmx(GT, gdiff)

    E_ref[0] = E
    F_ref[0] = -gR
    Q_ref[0] = Q
    Bm_ref[0] = Bmm
    D_ref[0] = Dm


def _run(interpret, R, Z, N,
         chg_embed, chg_Wf1, chg_bf1, chg_Wf2, chg_bf2, chg_Wu, chg_bu,
         chg_Wa, chg_Wp,
         dlt_embed, dlt_Wf1, dlt_bf1, dlt_Wf2, dlt_bf2, dlt_Wu, dlt_bu,
         dlt_Wa, dlt_Wp):
    Zc = Z.reshape(B_, A_, 1).astype(jnp.int32)
    N3 = N.reshape(B_, A_, NN_).astype(jnp.int32)
    Nrow = N.reshape(B_, 1, AN).astype(jnp.int32)

    def st(c, d):
        return jnp.stack([c, d]).astype(f32)

    embed2 = st(chg_embed, dlt_embed)
    Wf12 = st(chg_Wf1, dlt_Wf1)
    bf12 = st(chg_bf1, dlt_bf1).reshape(2, T_, 1, F_)
    Wf22 = st(chg_Wf2, dlt_Wf2)
    bf22 = st(chg_bf2, dlt_bf2).reshape(2, T_, 1, F_)
    Wu2 = st(chg_Wu, dlt_Wu)
    bu2 = st(chg_bu, dlt_bu).reshape(2, T_, 1, F_)
    Wa2 = st(chg_Wa, dlt_Wa)
    Wp2 = st(chg_Wp, dlt_Wp)
    Wf1T2 = jnp.transpose(Wf12, (0, 1, 3, 2))
    Wf2T2 = jnp.transpose(Wf22, (0, 1, 3, 2))
    WuT2 = jnp.transpose(Wu2, (0, 1, 3, 2))
    WaT2 = Wa2.reshape(2, 1, F_)
    WpT2 = Wp2.reshape(2, 1, F_)

    wargs = (embed2, Wf12, bf12, Wf22, bf22, Wu2, bu2, Wa2, Wp2,
             Wf1T2, Wf2T2, WuT2, WaT2, WpT2)

    def bspec(shape):
        return pl.BlockSpec((1,) + shape, lambda b: (b, 0, 0))

    def wspec(arr):
        nd = arr.ndim
        return pl.BlockSpec(arr.shape, lambda b, _n=nd: (0,) * _n)

    in_specs = [bspec((A_, 3)), bspec((A_, 1)), bspec((A_, NN_)), bspec((1, AN))]
    in_specs += [wspec(a) for a in wargs]

    out_shapes = (jax.ShapeDtypeStruct((B_, 1, 1), f32),
                  jax.ShapeDtypeStruct((B_, A_, 3), f32),
                  jax.ShapeDtypeStruct((B_, A_, 1), f32),
                  jax.ShapeDtypeStruct((B_, A_, NN_), f32),
                  jax.ShapeDtypeStruct((B_, A_, NN_), f32))
    out_specs = (bspec((1, 1)), bspec((A_, 3)), bspec((A_, 1)),
                 bspec((A_, NN_)), bspec((A_, NN_)))

    scratch = [pltpu.VMEM((2, T_ + 1, A_, F_), f32)]

    E3, F, Q, Bm, D = pl.pallas_call(
        _mayer_body,
        grid=(B_,),
        in_specs=in_specs,
        out_specs=out_specs,
        out_shape=out_shapes,
        scratch_shapes=scratch,
        compiler_params=pltpu.CompilerParams(
            dimension_semantics=("parallel",)),
        interpret=interpret,
    )(R.astype(f32), Zc, N3, Nrow, *wargs)

    return (E3.reshape(B_, 1), F, Q, Bm, D)


def kernel(R, Z, N,
           chg_embed, chg_Wf1, chg_bf1, chg_Wf2, chg_bf2, chg_Wu, chg_bu,
           chg_Wa, chg_Wp,
           dlt_embed, dlt_Wf1, dlt_bf1, dlt_Wf2, dlt_bf2, dlt_Wu, dlt_bu,
           dlt_Wa, dlt_Wp):
    return _run(False, R, Z, N,
                chg_embed, chg_Wf1, chg_bf1, chg_Wf2, chg_bf2, chg_Wu,
                chg_bu, chg_Wa, chg_Wp,
                dlt_embed, dlt_Wf1, dlt_bf1, dlt_Wf2, dlt_bf2, dlt_Wu,
                dlt_bu, dlt_Wa, dlt_Wp)


# speedup vs baseline: 1.3309x; 1.1829x over previous
import jax
import jax.numpy as jnp
from jax.experimental import pallas as pl
from jax.experimental.pallas import tpu as pltpu

B_, A_, NN_, F_, RES_, T_ = 16, 128, 32, 128, 20, 3
CUTOFF = 5.0
K_COUL = 332.063711
AN = A_ * NN_
f32 = jnp.float32


def _sig(x):
    return jax.nn.sigmoid(x)


def _silu(x):
    return x * _sig(x)


def _dsilu(x):
    s = _sig(x)
    return s * (1.0 + x * (1.0 - s))


def _mm(a, b):
    return jax.lax.dot(a, b, preferred_element_type=f32)


def _mmx(a, b):
    bf16 = jnp.bfloat16
    hi = b.astype(bf16).astype(f32)
    r1 = b - hi
    mid = r1.astype(bf16).astype(f32)
    lo = (r1 - mid).astype(bf16).astype(f32)
    return _mm(a, hi) + _mm(a, mid) + _mm(a, lo)


def _mayer_body(R_ref, Zc_ref, N3_ref, Nrow_ref,
                embed_r, Wf1_r, bf1_r, Wf2_r, bf2_r, Wu_r, bu_r,
                Wa_r, Wp_r, Wf1T_r, Wf2T_r, WuT_r, WaT_r, WpT_r,
                E_ref, F_ref, Q_ref, Bm_ref, D_ref,
                h3_ref, hj_ref, W_ref, dsg_ref, u_ref):
    R = R_ref[0]
    Zc = Zc_ref[0]
    N3 = N3_ref[0]
    Nrow = Nrow_ref[0]

    iota3 = jax.lax.broadcasted_iota(jnp.int32, (A_, NN_, A_), 2)
    G = (N3[:, :, None] == iota3).astype(f32).reshape(AN, A_)
    row_a = jax.lax.broadcasted_iota(jnp.int32, (A_, AN), 0)
    GT = (Nrow == row_a).astype(f32)

    def _rep(x):
        w = x.shape[1]
        return jnp.broadcast_to(x[:, None, :], (A_, NN_, w)).reshape(AN, w)

    def _seg(x):
        w = x.shape[1]
        return jnp.sum(x.reshape(A_, NN_, w), axis=1)

    eye3 = (jax.lax.broadcasted_iota(jnp.int32, (1, NN_, NN_), 1) ==
            jax.lax.broadcasted_iota(jnp.int32, (1, NN_, NN_), 2)).astype(f32)

    def _colify(xm):
        return jnp.sum(xm[:, None, :] * eye3, axis=2,
                       keepdims=True).reshape(AN, 1)

    def _matify(xc):
        return jnp.sum(xc.reshape(A_, NN_, 1) * eye3, axis=1)

    D2 = jnp.sum((_rep(R) - _mmx(G, R)) ** 2, axis=1, keepdims=True)
    D = jnp.sqrt(D2 + 1e-12)
    Dm = _matify(D)
    centers = (jax.lax.broadcasted_iota(jnp.int32, (1, RES_), 1).astype(f32)
               * (CUTOFF / (RES_ - 1)))
    rbf = jnp.exp(-10.0 * (D - centers) ** 2)
    fcm = 0.5 * (jnp.cos(jnp.pi * jnp.clip(Dm / CUTOFF, 0.0, 1.0)) + 1.0) \
        * (Dm < CUTOFF).astype(f32)
    fc = _colify(fcm)

    lane_z = jax.lax.broadcasted_iota(jnp.int32, (A_, 100), 1)
    onehotZ = (Zc == lane_z).astype(f32)

    bf16 = jnp.bfloat16

    def fwd_net(inet, _):
        h = _mm(onehotZ, embed_r[inet])

        def layer(t, h):
            hj = _mm(G, h)
            pre = _mm(rbf, Wf1_r[inet, t]) + bf1_r[inet, t]
            sg = _sig(pre)
            W = _mm(pre * sg, Wf2_r[inet, t]) + bf2_r[inet, t]
            m = _seg(hj * (W * fc))
            u = _mm(m, Wu_r[inet, t]) + bu_r[inet, t]
            hj_ref[inet, t] = hj.astype(bf16)
            W_ref[inet, t] = W.astype(bf16)
            dsg_ref[inet, t] = (sg * (1.0 + pre * (1.0 - sg))).astype(bf16)
            u_ref[inet, t] = u
            return h + _silu(u)

        h = jax.lax.fori_loop(0, T_, layer, h)
        h3_ref[inet] = h
        return 0

    jax.lax.fori_loop(0, 2, fwd_net, 0)

    h3c = h3_ref[0]
    h3d = h3_ref[1]
    Q = _mm(h3c, Wa_r[0])
    Bm = _mm(_rep(h3c) * _mm(G, h3c), Wp_r[0])

    qim = jnp.broadcast_to(Q, (A_, NN_))
    qjm = _matify(_mmx(G, Q))
    Bmm = _matify(Bm)
    maskm = (Dm > 1e-6).astype(f32)
    Dinvm = maskm * (1.0 / Dm)
    E_coul = 0.5 * K_COUL * jnp.sum(Dinvm * qim * qjm, axis=(0, 1), keepdims=True)
    E_bond = -0.25 * K_COUL * jnp.sum(Dinvm * Bmm * Bmm, axis=(0, 1), keepdims=True)
    dE = jnp.sum(_mm(h3d, Wa_r[1]), axis=(0, 1), keepdims=True)
    E = E_coul + E_bond + dE

    gQ = 0.5 * K_COUL * (jnp.sum(Dinvm * qjm, axis=1, keepdims=True)
                         + _mmx(GT, _colify(Dinvm * qim)))
    gBm = _colify(-0.5 * K_COUL * Dinvm * Bmm)
    gDinvm = 0.5 * K_COUL * qim * qjm - 0.25 * K_COUL * Bmm * Bmm
    gDm = -gDinvm * Dinvm * Dinvm * maskm

    ones_A1 = jnp.ones((A_, 1), f32)
    zeros_AN1 = jnp.zeros((AN, 1), f32)

    def bwd_net(inet, carry):
        grbf_t, gfc_t = carry
        is_chg = (inet == 0)
        gAi = jnp.where(is_chg, gQ, ones_A1)
        gPij = jnp.where(is_chg, gBm, zeros_AN1)
        h3 = h3_ref[inet]
        hj3 = _mm(G, h3)
        hrep = _rep(h3)
        gh0 = gAi * WaT_r[inet]
        WpT = WpT_r[inet]
        ghrep = gPij * (hj3 * WpT)
        ghj3 = gPij * (hrep * WpT)
        gh0 = gh0 + _seg(ghrep) + _mm(GT, ghj3)

        def layer(i, carry):
            gh, grbf, gfc = carry
            t = T_ - 1 - i
            hj = hj_ref[inet, t]
            W = W_ref[inet, t]
            u = u_ref[inet, t]
            gu = gh * _dsilu(u)
            gm = _mm(gu, WuT_r[inet, t])
            gmr = _rep(gm)
            gmrfc = gmr * fc
            t1 = gmr * hj
            ghj = gmrfc * W
            gW = t1 * fc
            gfc = gfc + jnp.sum(t1 * W, axis=1, keepdims=True)
            gpre = _mm(gW, Wf2T_r[inet, t]) * dsg_ref[inet, t]
            grbf = grbf + _mm(gpre, Wf1T_r[inet, t])
            gh = gh + _mm(GT, ghj)
            return gh, grbf, gfc

        _, grbf_t, gfc_t = jax.lax.fori_loop(
            0, T_, layer, (gh0, grbf_t, gfc_t))
        return grbf_t, gfc_t

    grbf, gfc = jax.lax.fori_loop(
        0, 2, bwd_net, (jnp.zeros((AN, RES_), f32), zeros_AN1))

    gD_rbf = jnp.sum(grbf * rbf * (-20.0 * (D - centers)), axis=1, keepdims=True)
    gDm = gDm + _matify(gD_rbf)
    gDm = gDm + _matify(gfc) * (-0.5 * jnp.pi / CUTOFF) * jnp.sin(
        jnp.pi * jnp.clip(Dm / CUTOFF, 0.0, 1.0)) * (Dm < CUTOFF).astype(f32)

    diff = _rep(R) - _mmx(G, R)
    gdiff = _colify(gDm / Dm) * diff
    gR = _seg(gdiff) - _mmx(GT, gdiff)

    E_ref[0] = E
    F_ref[0] = -gR
    Q_ref[0] = Q
    Bm_ref[0] = Bmm
    D_ref[0] = Dm


def _run(interpret, R, Z, N,
         chg_embed, chg_Wf1, chg_bf1, chg_Wf2, chg_bf2, chg_Wu, chg_bu,
         chg_Wa, chg_Wp,
         dlt_embed, dlt_Wf1, dlt_bf1, dlt_Wf2, dlt_bf2, dlt_Wu, dlt_bu,
         dlt_Wa, dlt_Wp):
    Zc = Z.reshape(B_, A_, 1).astype(jnp.int32)
    N3 = N.reshape(B_, A_, NN_).astype(jnp.int32)
    Nrow = N.reshape(B_, 1, AN).astype(jnp.int32)

    def st(c, d):
        return jnp.stack([c, d]).astype(f32)

    embed2 = st(chg_embed, dlt_embed)
    Wf12 = st(chg_Wf1, dlt_Wf1)
    bf12 = st(chg_bf1, dlt_bf1).reshape(2, T_, 1, F_)
    Wf22 = st(chg_Wf2, dlt_Wf2)
    bf22 = st(chg_bf2, dlt_bf2).reshape(2, T_, 1, F_)
    Wu2 = st(chg_Wu, dlt_Wu)
    bu2 = st(chg_bu, dlt_bu).reshape(2, T_, 1, F_)
    Wa2 = st(chg_Wa, dlt_Wa)
    Wp2 = st(chg_Wp, dlt_Wp)
    Wf1T2 = jnp.transpose(Wf12, (0, 1, 3, 2))
    Wf2T2 = jnp.transpose(Wf22, (0, 1, 3, 2))
    WuT2 = jnp.transpose(Wu2, (0, 1, 3, 2))
    WaT2 = Wa2.reshape(2, 1, F_)
    WpT2 = Wp2.reshape(2, 1, F_)

    wargs = (embed2, Wf12, bf12, Wf22, bf22, Wu2, bu2, Wa2, Wp2,
             Wf1T2, Wf2T2, WuT2, WaT2, WpT2)

    def bspec(shape):
        return pl.BlockSpec((1,) + shape, lambda b: (b, 0, 0))

    def wspec(arr):
        nd = arr.ndim
        return pl.BlockSpec(arr.shape, lambda b, _n=nd: (0,) * _n)

    in_specs = [bspec((A_, 3)), bspec((A_, 1)), bspec((A_, NN_)), bspec((1, AN))]
    in_specs += [wspec(a) for a in wargs]

    out_shapes = (jax.ShapeDtypeStruct((B_, 1, 1), f32),
                  jax.ShapeDtypeStruct((B_, A_, 3), f32),
                  jax.ShapeDtypeStruct((B_, A_, 1), f32),
                  jax.ShapeDtypeStruct((B_, A_, NN_), f32),
                  jax.ShapeDtypeStruct((B_, A_, NN_), f32))
    out_specs = (bspec((1, 1)), bspec((A_, 3)), bspec((A_, 1)),
                 bspec((A_, NN_)), bspec((A_, NN_)))

    scratch = [pltpu.VMEM((2, A_, F_), f32),
               pltpu.VMEM((2, T_, AN, F_), jnp.bfloat16),
               pltpu.VMEM((2, T_, AN, F_), jnp.bfloat16),
               pltpu.VMEM((2, T_, AN, F_), jnp.bfloat16),
               pltpu.VMEM((2, T_, A_, F_), f32)]

    E3, F, Q, Bm, D = pl.pallas_call(
        _mayer_body,
        grid=(B_,),
        in_specs=in_specs,
        out_specs=out_specs,
        out_shape=out_shapes,
        scratch_shapes=scratch,
        compiler_params=pltpu.CompilerParams(
            dimension_semantics=("parallel",)),
        interpret=interpret,
    )(R.astype(f32), Zc, N3, Nrow, *wargs)

    return (E3.reshape(B_, 1), F, Q, Bm, D)


def kernel(R, Z, N,
           chg_embed, chg_Wf1, chg_bf1, chg_Wf2, chg_bf2, chg_Wu, chg_bu,
           chg_Wa, chg_Wp,
           dlt_embed, dlt_Wf1, dlt_bf1, dlt_Wf2, dlt_bf2, dlt_Wu, dlt_bu,
           dlt_Wa, dlt_Wp):
    return _run(False, R, Z, N,
                chg_embed, chg_Wf1, chg_bf1, chg_Wf2, chg_bf2, chg_Wu,
                chg_bu, chg_Wa, chg_Wp,
                dlt_embed, dlt_Wf1, dlt_bf1, dlt_Wf2, dlt_bf2, dlt_Wu,
                dlt_bu, dlt_Wa, dlt_Wp)


# speedup vs baseline: 1.3791x; 1.0362x over previous
import jax
import jax.numpy as jnp
from jax.experimental import pallas as pl
from jax.experimental.pallas import tpu as pltpu

B_, A_, NN_, F_, RES_, T_ = 16, 128, 32, 128, 20, 3
CUTOFF = 5.0
K_COUL = 332.063711
AN = A_ * NN_
f32 = jnp.float32


def _sig(x):
    return jax.nn.sigmoid(x)


def _silu(x):
    return x * _sig(x)


def _dsilu(x):
    s = _sig(x)
    return s * (1.0 + x * (1.0 - s))


def _cos_pi01(t):
    u = 2.0 * t - 1.0
    u2 = u * u
    p = 1.5707963267948966 + u2 * (-0.6459640975062462 + u2 * (
        0.07969262624616703 + u2 * (-0.004681754135318687 + u2 * (
            0.00016044118478735982 - u2 * 3.598843235212055e-06))))
    return -u * p


def _sin_pi01(t):
    x2 = (jnp.pi * (t - 0.5)) ** 2
    return 1.0 + x2 * (-0.5 + x2 * (1.0 / 24.0 + x2 * (
        -1.0 / 720.0 + x2 * (1.0 / 40320.0 - x2 * (1.0 / 3628800.0)))))


def _mm(a, b):
    return jax.lax.dot(a, b, preferred_element_type=f32)


def _mmx(a, b):
    bf16 = jnp.bfloat16
    hi = b.astype(bf16).astype(f32)
    r1 = b - hi
    mid = r1.astype(bf16).astype(f32)
    lo = (r1 - mid).astype(bf16).astype(f32)
    return _mm(a, hi) + _mm(a, mid) + _mm(a, lo)


def _mayer_body(R_ref, Zc_ref, N3_ref, Nrow_ref,
                embed_r, Wf1_r, bf1_r, Wf2_r, bf2_r, Wu_r, bu_r,
                Wa_r, Wp_r, Wf1T_r, Wf2T_r, WuT_r, WaT_r, WpT_r,
                E_ref, F_ref, Q_ref, Bm_ref, D_ref,
                h3_ref, hj_ref, W_ref, dsg_ref, u_ref):
    R = R_ref[0]
    Zc = Zc_ref[0]
    N3 = N3_ref[0]
    Nrow = Nrow_ref[0]

    iota3 = jax.lax.broadcasted_iota(jnp.int32, (A_, NN_, A_), 2)
    G = (N3[:, :, None] == iota3).astype(f32).reshape(AN, A_)
    row_a = jax.lax.broadcasted_iota(jnp.int32, (A_, AN), 0)
    GT = (Nrow == row_a).astype(f32)

    def _rep(x):
        w = x.shape[1]
        return jnp.broadcast_to(x[:, None, :], (A_, NN_, w)).reshape(AN, w)

    def _seg(x):
        w = x.shape[1]
        return jnp.sum(x.reshape(A_, NN_, w), axis=1)

    eye3 = (jax.lax.broadcasted_iota(jnp.int32, (1, NN_, NN_), 1) ==
            jax.lax.broadcasted_iota(jnp.int32, (1, NN_, NN_), 2)).astype(f32)

    def _colify(xm):
        return jnp.sum(xm[:, None, :] * eye3, axis=2,
                       keepdims=True).reshape(AN, 1)

    def _matify(xc):
        return jnp.sum(xc.reshape(A_, NN_, 1) * eye3, axis=1)

    D2 = jnp.sum((_rep(R) - _mmx(G, R)) ** 2, axis=1, keepdims=True)
    D = jnp.sqrt(D2 + 1e-12)
    Dm = _matify(D)
    centers = (jax.lax.broadcasted_iota(jnp.int32, (1, RES_), 1).astype(f32)
               * (CUTOFF / (RES_ - 1)))
    rbf = jnp.exp(-10.0 * (D - centers) ** 2)
    fcm = 0.5 * (_cos_pi01(jnp.clip(Dm / CUTOFF, 0.0, 1.0)) + 1.0) \
        * (Dm < CUTOFF).astype(f32)
    fc = _colify(fcm)

    lane_z = jax.lax.broadcasted_iota(jnp.int32, (A_, 100), 1)
    onehotZ = (Zc == lane_z).astype(f32)

    bf16 = jnp.bfloat16

    def fwd_net(inet, _):
        h = _mm(onehotZ, embed_r[inet])

        def layer(t, h):
            hj = _mm(G, h)
            pre = _mm(rbf, Wf1_r[inet, t]) + bf1_r[inet, t]
            sg = _sig(pre)
            W = _mm(pre * sg, Wf2_r[inet, t]) + bf2_r[inet, t]
            m = _seg(hj * (W * fc))
            u = _mm(m, Wu_r[inet, t]) + bu_r[inet, t]
            hj_ref[inet, t] = hj.astype(bf16)
            W_ref[inet, t] = W.astype(bf16)
            dsg_ref[inet, t] = (sg * (1.0 + pre * (1.0 - sg))).astype(bf16)
            u_ref[inet, t] = u
            return h + _silu(u)

        h = jax.lax.fori_loop(0, T_, layer, h)
        h3_ref[inet] = h
        return 0

    jax.lax.fori_loop(0, 2, fwd_net, 0)

    h3c = h3_ref[0]
    h3d = h3_ref[1]
    Q = _mm(h3c, Wa_r[0])
    Bm = _mm(_rep(h3c) * _mm(G, h3c), Wp_r[0])

    qim = jnp.broadcast_to(Q, (A_, NN_))
    qjm = _matify(_mmx(G, Q))
    Bmm = _matify(Bm)
    maskm = (Dm > 1e-6).astype(f32)
    Dinvm = maskm * (1.0 / Dm)
    E_coul = 0.5 * K_COUL * jnp.sum(Dinvm * qim * qjm, axis=(0, 1), keepdims=True)
    E_bond = -0.25 * K_COUL * jnp.sum(Dinvm * Bmm * Bmm, axis=(0, 1), keepdims=True)
    dE = jnp.sum(_mm(h3d, Wa_r[1]), axis=(0, 1), keepdims=True)
    E = E_coul + E_bond + dE

    gQ = 0.5 * K_COUL * (jnp.sum(Dinvm * qjm, axis=1, keepdims=True)
                         + _mmx(GT, _colify(Dinvm * qim)))
    gBm = _colify(-0.5 * K_COUL * Dinvm * Bmm)
    gDinvm = 0.5 * K_COUL * qim * qjm - 0.25 * K_COUL * Bmm * Bmm
    gDm = -gDinvm * Dinvm * Dinvm * maskm

    ones_A1 = jnp.ones((A_, 1), f32)
    zeros_AN1 = jnp.zeros((AN, 1), f32)

    def bwd_net(inet, carry):
        grbf_t, gfc_t = carry
        is_chg = (inet == 0)
        gAi = jnp.where(is_chg, gQ, ones_A1)
        gPij = jnp.where(is_chg, gBm, zeros_AN1)
        h3 = h3_ref[inet]
        hj3 = _mm(G, h3)
        hrep = _rep(h3)
        gh0 = gAi * WaT_r[inet]
        WpT = WpT_r[inet]
        ghrep = gPij * (hj3 * WpT)
        ghj3 = gPij * (hrep * WpT)
        gh0 = gh0 + _seg(ghrep) + _mm(GT, ghj3)

        def layer(i, carry):
            gh, grbf, gfc = carry
            t = T_ - 1 - i
            hj = hj_ref[inet, t]
            W = W_ref[inet, t]
            u = u_ref[inet, t]
            gu = gh * _dsilu(u)
            gm = _mm(gu, WuT_r[inet, t])
            gmr = _rep(gm)
            gmrfc = gmr * fc
            t1 = gmr * hj
            ghj = gmrfc * W
            gW = t1 * fc
            gfc = gfc + jnp.sum(t1 * W, axis=1, keepdims=True)
            gpre = _mm(gW, Wf2T_r[inet, t]) * dsg_ref[inet, t]
            grbf = grbf + _mm(gpre, Wf1T_r[inet, t])
            gh = gh + _mm(GT, ghj)
            return gh, grbf, gfc

        _, grbf_t, gfc_t = jax.lax.fori_loop(
            0, T_, layer, (gh0, grbf_t, gfc_t))
        return grbf_t, gfc_t

    grbf, gfc = jax.lax.fori_loop(
        0, 2, bwd_net, (jnp.zeros((AN, RES_), f32), zeros_AN1))

    gD_rbf = jnp.sum(grbf * rbf * (-20.0 * (D - centers)), axis=1, keepdims=True)
    gDm = gDm + _matify(gD_rbf)
    gDm = gDm + _matify(gfc) * (-0.5 * jnp.pi / CUTOFF) * _sin_pi01(
        jnp.clip(Dm / CUTOFF, 0.0, 1.0)) * (Dm < CUTOFF).astype(f32)

    diff = _rep(R) - _mmx(G, R)
    gdiff = _colify(gDm / Dm) * diff
    gR = _seg(gdiff) - _mmx(GT, gdiff)

    E_ref[0] = E
    F_ref[0] = -gR
    Q_ref[0] = Q
    Bm_ref[0] = Bmm
    D_ref[0] = Dm


def _run(interpret, R, Z, N,
         chg_embed, chg_Wf1, chg_bf1, chg_Wf2, chg_bf2, chg_Wu, chg_bu,
         chg_Wa, chg_Wp,
         dlt_embed, dlt_Wf1, dlt_bf1, dlt_Wf2, dlt_bf2, dlt_Wu, dlt_bu,
         dlt_Wa, dlt_Wp):
    Zc = Z.reshape(B_, A_, 1).astype(jnp.int32)
    N3 = N.reshape(B_, A_, NN_).astype(jnp.int32)
    Nrow = N.reshape(B_, 1, AN).astype(jnp.int32)

    def st(c, d):
        return jnp.stack([c, d]).astype(f32)

    embed2 = st(chg_embed, dlt_embed)
    Wf12 = st(chg_Wf1, dlt_Wf1)
    bf12 = st(chg_bf1, dlt_bf1).reshape(2, T_, 1, F_)
    Wf22 = st(chg_Wf2, dlt_Wf2)
    bf22 = st(chg_bf2, dlt_bf2).reshape(2, T_, 1, F_)
    Wu2 = st(chg_Wu, dlt_Wu)
    bu2 = st(chg_bu, dlt_bu).reshape(2, T_, 1, F_)
    Wa2 = st(chg_Wa, dlt_Wa)
    Wp2 = st(chg_Wp, dlt_Wp)
    Wf1T2 = jnp.transpose(Wf12, (0, 1, 3, 2))
    Wf2T2 = jnp.transpose(Wf22, (0, 1, 3, 2))
    WuT2 = jnp.transpose(Wu2, (0, 1, 3, 2))
    WaT2 = Wa2.reshape(2, 1, F_)
    WpT2 = Wp2.reshape(2, 1, F_)

    wargs = (embed2, Wf12, bf12, Wf22, bf22, Wu2, bu2, Wa2, Wp2,
             Wf1T2, Wf2T2, WuT2, WaT2, WpT2)

    def bspec(shape):
        return pl.BlockSpec((1,) + shape, lambda b: (b, 0, 0))

    def wspec(arr):
        nd = arr.ndim
        return pl.BlockSpec(arr.shape, lambda b, _n=nd: (0,) * _n)

    in_specs = [bspec((A_, 3)), bspec((A_, 1)), bspec((A_, NN_)), bspec((1, AN))]
    in_specs += [wspec(a) for a in wargs]

    out_shapes = (jax.ShapeDtypeStruct((B_, 1, 1), f32),
                  jax.ShapeDtypeStruct((B_, A_, 3), f32),
                  jax.ShapeDtypeStruct((B_, A_, 1), f32),
                  jax.ShapeDtypeStruct((B_, A_, NN_), f32),
                  jax.ShapeDtypeStruct((B_, A_, NN_), f32))
    out_specs = (bspec((1, 1)), bspec((A_, 3)), bspec((A_, 1)),
                 bspec((A_, NN_)), bspec((A_, NN_)))

    scratch = [pltpu.VMEM((2, A_, F_), f32),
               pltpu.VMEM((2, T_, AN, F_), jnp.bfloat16),
               pltpu.VMEM((2, T_, AN, F_), jnp.bfloat16),
               pltpu.VMEM((2, T_, AN, F_), jnp.bfloat16),
               pltpu.VMEM((2, T_, A_, F_), f32)]

    E3, F, Q, Bm, D = pl.pallas_call(
        _mayer_body,
        grid=(B_,),
        in_specs=in_specs,
        out_specs=out_specs,
        out_shape=out_shapes,
        scratch_shapes=scratch,
        compiler_params=pltpu.CompilerParams(
            dimension_semantics=("parallel",)),
        interpret=interpret,
    )(R.astype(f32), Zc, N3, Nrow, *wargs)

    return (E3.reshape(B_, 1), F, Q, Bm, D)


def kernel(R, Z, N,
           chg_embed, chg_Wf1, chg_bf1, chg_Wf2, chg_bf2, chg_Wu, chg_bu,
           chg_Wa, chg_Wp,
           dlt_embed, dlt_Wf1, dlt_bf1, dlt_Wf2, dlt_bf2, dlt_Wu, dlt_bu,
           dlt_Wa, dlt_Wp):
    return _run(False, R, Z, N,
                chg_embed, chg_Wf1, chg_bf1, chg_Wf2, chg_bf2, chg_Wu,
                chg_bu, chg_Wa, chg_Wp,
                dlt_embed, dlt_Wf1, dlt_bf1, dlt_Wf2, dlt_bf2, dlt_Wu,
                dlt_bu, dlt_Wa, dlt_Wp)


# speedup vs baseline: 1.3902x; 1.0080x over previous
import jax
import jax.numpy as jnp
from jax.experimental import pallas as pl
from jax.experimental.pallas import tpu as pltpu

B_, A_, NN_, F_, RES_, T_ = 16, 128, 32, 128, 20, 3
CUTOFF = 5.0
K_COUL = 332.063711
AN = A_ * NN_
f32 = jnp.float32


def _sig(x):
    return jax.nn.sigmoid(x)


def _silu(x):
    return x * _sig(x)


def _dsilu(x):
    s = _sig(x)
    return s * (1.0 + x * (1.0 - s))


def _cos_pi01(t):
    u = 2.0 * t - 1.0
    u2 = u * u
    p = 1.5707963267948966 + u2 * (-0.6459640975062462 + u2 * (
        0.07969262624616703 + u2 * (-0.004681754135318687 + u2 * (
            0.00016044118478735982 - u2 * 3.598843235212055e-06))))
    return -u * p


def _sin_pi01(t):
    x2 = (jnp.pi * (t - 0.5)) ** 2
    return 1.0 + x2 * (-0.5 + x2 * (1.0 / 24.0 + x2 * (
        -1.0 / 720.0 + x2 * (1.0 / 40320.0 - x2 * (1.0 / 3628800.0)))))


def _mm(a, b):
    return jax.lax.dot(a, b, preferred_element_type=f32)


def _mmx(a, b):
    bf16 = jnp.bfloat16
    hi = b.astype(bf16)
    r1 = b - hi.astype(f32)
    mid = r1.astype(bf16)
    lo = (r1 - mid.astype(f32)).astype(bf16)
    return _mm(a, hi) + _mm(a, mid) + _mm(a, lo)


def _mayer_body(R_ref, Zc_ref, N3_ref, Nrow_ref,
                embed_r, Wf1_r, bf1_r, Wf2_r, bf2_r, Wu_r, bu_r,
                Wa_r, Wp_r, Wf1T_r, Wf2T_r, WuT_r, WaT_r, WpT_r,
                E_ref, F_ref, Q_ref, Bm_ref, D_ref,
                h3_ref, hj_ref, W_ref, dsg_ref, u_ref):
    R = R_ref[0]
    Zc = Zc_ref[0]
    N3 = N3_ref[0]
    Nrow = Nrow_ref[0]

    iota3 = jax.lax.broadcasted_iota(jnp.int32, (A_, NN_, A_), 2)
    G = (N3[:, :, None] == iota3).astype(jnp.bfloat16).reshape(AN, A_)
    row_a = jax.lax.broadcasted_iota(jnp.int32, (A_, AN), 0)
    GT = (Nrow == row_a).astype(jnp.bfloat16)

    def _mg(sel, x):
        return jax.lax.dot(sel, x.astype(jnp.bfloat16),
                           preferred_element_type=f32)

    def _rep(x):
        w = x.shape[1]
        return jnp.broadcast_to(x[:, None, :], (A_, NN_, w)).reshape(AN, w)

    def _seg(x):
        w = x.shape[1]
        return jnp.sum(x.reshape(A_, NN_, w), axis=1)

    eye3 = (jax.lax.broadcasted_iota(jnp.int32, (1, NN_, NN_), 1) ==
            jax.lax.broadcasted_iota(jnp.int32, (1, NN_, NN_), 2)).astype(f32)

    def _colify(xm):
        return jnp.sum(xm[:, None, :] * eye3, axis=2,
                       keepdims=True).reshape(AN, 1)

    def _matify(xc):
        return jnp.sum(xc.reshape(A_, NN_, 1) * eye3, axis=1)

    D2 = jnp.sum((_rep(R) - _mmx(G, R)) ** 2, axis=1, keepdims=True)
    D = jnp.sqrt(D2 + 1e-12)
    Dm = _matify(D)
    centers = (jax.lax.broadcasted_iota(jnp.int32, (1, RES_), 1).astype(f32)
               * (CUTOFF / (RES_ - 1)))
    rbf = jnp.exp(-10.0 * (D - centers) ** 2)
    fcm = 0.5 * (_cos_pi01(jnp.clip(Dm / CUTOFF, 0.0, 1.0)) + 1.0) \
        * (Dm < CUTOFF).astype(f32)
    fc = _colify(fcm)

    lane_z = jax.lax.broadcasted_iota(jnp.int32, (A_, 100), 1)
    onehotZ = (Zc == lane_z).astype(f32)

    bf16 = jnp.bfloat16

    def fwd_net(inet, _):
        h = _mm(onehotZ, embed_r[inet])

        def layer(t, h):
            hj = _mg(G, h)
            pre = _mm(rbf, Wf1_r[inet, t]) + bf1_r[inet, t]
            sg = _sig(pre)
            W = _mm(pre * sg, Wf2_r[inet, t]) + bf2_r[inet, t]
            m = _seg(hj * (W * fc))
            u = _mm(m, Wu_r[inet, t]) + bu_r[inet, t]
            hj_ref[inet, t] = hj.astype(bf16)
            W_ref[inet, t] = W.astype(bf16)
            dsg_ref[inet, t] = (sg * (1.0 + pre * (1.0 - sg))).astype(bf16)
            u_ref[inet, t] = u
            return h + _silu(u)

        h = jax.lax.fori_loop(0, T_, layer, h)
        h3_ref[inet] = h
        return 0

    jax.lax.fori_loop(0, 2, fwd_net, 0)

    h3c = h3_ref[0]
    h3d = h3_ref[1]
    Q = _mm(h3c, Wa_r[0])
    Bm = _mm(_rep(h3c) * _mg(G, h3c), Wp_r[0])

    qim = jnp.broadcast_to(Q, (A_, NN_))
    qjm = _matify(_mmx(G, Q))
    Bmm = _matify(Bm)
    maskm = (Dm > 1e-6).astype(f32)
    Dinvm = maskm * (1.0 / Dm)
    E_coul = 0.5 * K_COUL * jnp.sum(Dinvm * qim * qjm, axis=(0, 1), keepdims=True)
    E_bond = -0.25 * K_COUL * jnp.sum(Dinvm * Bmm * Bmm, axis=(0, 1), keepdims=True)
    dE = jnp.sum(_mm(h3d, Wa_r[1]), axis=(0, 1), keepdims=True)
    E = E_coul + E_bond + dE

    gQ = 0.5 * K_COUL * (jnp.sum(Dinvm * qjm, axis=1, keepdims=True)
                         + _mmx(GT, _colify(Dinvm * qim)))
    gBm = _colify(-0.5 * K_COUL * Dinvm * Bmm)
    gDinvm = 0.5 * K_COUL * qim * qjm - 0.25 * K_COUL * Bmm * Bmm
    gDm = -gDinvm * Dinvm * Dinvm * maskm

    ones_A1 = jnp.ones((A_, 1), f32)
    zeros_AN1 = jnp.zeros((AN, 1), f32)

    def bwd_net(inet, carry):
        grbf_t, gfc_t = carry
        is_chg = (inet == 0)
        gAi = jnp.where(is_chg, gQ, ones_A1)
        gPij = jnp.where(is_chg, gBm, zeros_AN1)
        h3 = h3_ref[inet]
        hj3 = _mg(G, h3)
        hrep = _rep(h3)
        gh0 = gAi * WaT_r[inet]
        WpT = WpT_r[inet]
        ghrep = gPij * (hj3 * WpT)
        ghj3 = gPij * (hrep * WpT)
        gh0 = gh0 + _seg(ghrep) + _mg(GT, ghj3)

        def layer(i, carry):
            gh, grbf, gfc = carry
            t = T_ - 1 - i
            hj = hj_ref[inet, t]
            W = W_ref[inet, t]
            u = u_ref[inet, t]
            gu = gh * _dsilu(u)
            gm = _mm(gu, WuT_r[inet, t])
            gmr = _rep(gm)
            gmrfc = gmr * fc
            t1 = gmr * hj
            ghj = gmrfc * W
            gW = t1 * fc
            gfc = gfc + jnp.sum(t1 * W, axis=1, keepdims=True)
            gpre = _mm(gW, Wf2T_r[inet, t]) * dsg_ref[inet, t]
            grbf = grbf + _mm(gpre, Wf1T_r[inet, t])
            gh = gh + _mg(GT, ghj)
            return gh, grbf, gfc

        _, grbf_t, gfc_t = jax.lax.fori_loop(
            0, T_, layer, (gh0, grbf_t, gfc_t))
        return grbf_t, gfc_t

    grbf, gfc = jax.lax.fori_loop(
        0, 2, bwd_net, (jnp.zeros((AN, RES_), f32), zeros_AN1))

    gD_rbf = jnp.sum(grbf * rbf * (-20.0 * (D - centers)), axis=1, keepdims=True)
    gDm = gDm + _matify(gD_rbf)
    gDm = gDm + _matify(gfc) * (-0.5 * jnp.pi / CUTOFF) * _sin_pi01(
        jnp.clip(Dm / CUTOFF, 0.0, 1.0)) * (Dm < CUTOFF).astype(f32)

    diff = _rep(R) - _mmx(G, R)
    gdiff = _colify(gDm / Dm) * diff
    gR = _seg(gdiff) - _mmx(GT, gdiff)

    E_ref[0] = E
    F_ref[0] = -gR
    Q_ref[0] = Q
    Bm_ref[0] = Bmm
    D_ref[0] = Dm


def _run(interpret, R, Z, N,
         chg_embed, chg_Wf1, chg_bf1, chg_Wf2, chg_bf2, chg_Wu, chg_bu,
         chg_Wa, chg_Wp,
         dlt_embed, dlt_Wf1, dlt_bf1, dlt_Wf2, dlt_bf2, dlt_Wu, dlt_bu,
         dlt_Wa, dlt_Wp):
    Zc = Z.reshape(B_, A_, 1).astype(jnp.int32)
    N3 = N.reshape(B_, A_, NN_).astype(jnp.int32)
    Nrow = N.reshape(B_, 1, AN).astype(jnp.int32)

    def st(c, d):
        return jnp.stack([c, d]).astype(f32)

    embed2 = st(chg_embed, dlt_embed)
    Wf12 = st(chg_Wf1, dlt_Wf1)
    bf12 = st(chg_bf1, dlt_bf1).reshape(2, T_, 1, F_)
    Wf22 = st(chg_Wf2, dlt_Wf2)
    bf22 = st(chg_bf2, dlt_bf2).reshape(2, T_, 1, F_)
    Wu2 = st(chg_Wu, dlt_Wu)
    bu2 = st(chg_bu, dlt_bu).reshape(2, T_, 1, F_)
    Wa2 = st(chg_Wa, dlt_Wa)
    Wp2 = st(chg_Wp, dlt_Wp)
    Wf1T2 = jnp.transpose(Wf12, (0, 1, 3, 2))
    Wf2T2 = jnp.transpose(Wf22, (0, 1, 3, 2))
    WuT2 = jnp.transpose(Wu2, (0, 1, 3, 2))
    WaT2 = Wa2.reshape(2, 1, F_)
    WpT2 = Wp2.reshape(2, 1, F_)

    wargs = (embed2, Wf12, bf12, Wf22, bf22, Wu2, bu2, Wa2, Wp2,
             Wf1T2, Wf2T2, WuT2, WaT2, WpT2)

    def bspec(shape):
        return pl.BlockSpec((1,) + shape, lambda b: (b, 0, 0))

    def wspec(arr):
        nd = arr.ndim
        return pl.BlockSpec(arr.shape, lambda b, _n=nd: (0,) * _n)

    in_specs = [bspec((A_, 3)), bspec((A_, 1)), bspec((A_, NN_)), bspec((1, AN))]
    in_specs += [wspec(a) for a in wargs]

    out_shapes = (jax.ShapeDtypeStruct((B_, 1, 1), f32),
                  jax.ShapeDtypeStruct((B_, A_, 3), f32),
                  jax.ShapeDtypeStruct((B_, A_, 1), f32),
                  jax.ShapeDtypeStruct((B_, A_, NN_), f32),
                  jax.ShapeDtypeStruct((B_, A_, NN_), f32))
    out_specs = (bspec((1, 1)), bspec((A_, 3)), bspec((A_, 1)),
                 bspec((A_, NN_)), bspec((A_, NN_)))

    scratch = [pltpu.VMEM((2, A_, F_), f32),
               pltpu.VMEM((2, T_, AN, F_), jnp.bfloat16),
               pltpu.VMEM((2, T_, AN, F_), jnp.bfloat16),
               pltpu.VMEM((2, T_, AN, F_), jnp.bfloat16),
               pltpu.VMEM((2, T_, A_, F_), f32)]

    E3, F, Q, Bm, D = pl.pallas_call(
        _mayer_body,
        grid=(B_,),
        in_specs=in_specs,
        out_specs=out_specs,
        out_shape=out_shapes,
        scratch_shapes=scratch,
        compiler_params=pltpu.CompilerParams(
            dimension_semantics=("parallel",)),
        interpret=interpret,
    )(R.astype(f32), Zc, N3, Nrow, *wargs)

    return (E3.reshape(B_, 1), F, Q, Bm, D)


def kernel(R, Z, N,
           chg_embed, chg_Wf1, chg_bf1, chg_Wf2, chg_bf2, chg_Wu, chg_bu,
           chg_Wa, chg_Wp,
           dlt_embed, dlt_Wf1, dlt_bf1, dlt_Wf2, dlt_bf2, dlt_Wu, dlt_bu,
           dlt_Wa, dlt_Wp):
    return _run(False, R, Z, N,
                chg_embed, chg_Wf1, chg_bf1, chg_Wf2, chg_bf2, chg_Wu,
                chg_bu, chg_Wa, chg_Wp,
                dlt_embed, dlt_Wf1, dlt_bf1, dlt_Wf2, dlt_bf2, dlt_Wu,
                dlt_bu, dlt_Wa, dlt_Wp)
